# Initial kernel scaffold; baseline (speedup 1.0000x reference)
import jax
import jax.numpy as jnp
from jax.experimental import pallas as pl


def kernel(tensor, table):
    raise NotImplementedError("write your pallas kernel here")


# speedup vs baseline: 1.5564x; 1.5564x over previous
import functools

import jax
import jax.numpy as jnp
from jax import lax
from jax.experimental import pallas as pl
from jax.experimental.pallas import tpu as pltpu
from jax.experimental.pallas import tpu_sc as plsc

_EMBED = 32
_VPG = 128
_CH = 8


def _flat_gather(idx2d, table):
    n_vec = idx2d.shape[0]
    info = plsc.get_sparse_core_info()
    nw = info.num_cores * info.num_subcores
    nvw = n_vec // nw
    nch = nvw // _CH
    rows_per_chunk = _CH * _VPG

    mesh = plsc.VectorSubcoreMesh(core_axis_name="c", subcore_axis_name="s")

    @functools.partial(
        pl.kernel,
        mesh=mesh,
        out_type=jax.ShapeDtypeStruct((n_vec * _VPG, _EMBED), jnp.float32),
        scratch_types=[
            pltpu.VMEM((nvw, _VPG), jnp.int32),
            pltpu.VMEM((rows_per_chunk, _EMBED), jnp.float32),
            pltpu.SemaphoreType.DMA,
        ],
        compiler_params=pltpu.CompilerParams(use_tc_tiling_on_sc=False),
    )
    def k(idx_hbm, table_hbm, out_hbm, idx_v, rows_v, sem):
        wid = lax.axis_index("s") * info.num_cores + lax.axis_index("c")
        vbase = wid * nvw
        pltpu.sync_copy(idx_hbm.at[pl.ds(vbase, nvw)], idx_v)

        def chunk(c, carry):
            cbase = c * _CH
            cps = [
                pltpu.async_copy(
                    table_hbm.at[idx_v.at[cbase + g]],
                    rows_v.at[pl.ds(g * _VPG, _VPG)],
                    sem,
                )
                for g in range(_CH)
            ]
            for cp in cps:
                cp.wait()
            pltpu.sync_copy(
                rows_v,
                out_hbm.at[pl.ds((vbase + cbase) * _VPG, rows_per_chunk)],
            )
            return carry

        lax.fori_loop(0, nch, chunk, 0)

    return k(idx2d, table)


def kernel(tensor, table):
    b, f = tensor.shape
    n = b * f
    idx2d = tensor.reshape(n // _VPG, _VPG).astype(jnp.int32)
    out = _flat_gather(idx2d, table)
    return out.reshape(b, f, _EMBED)


# speedup vs baseline: 1.5763x; 1.0127x over previous
import functools

import jax
import jax.numpy as jnp
from jax import lax
from jax.experimental import pallas as pl
from jax.experimental.pallas import tpu as pltpu
from jax.experimental.pallas import tpu_sc as plsc

_EMBED = 32
_RPC = 1664


def _flat_gather(idx, table):
    n = idx.shape[0]
    info = plsc.get_sparse_core_info()
    nw = info.num_cores * info.num_subcores
    npw = n // nw
    nch = npw // _RPC

    mesh = plsc.VectorSubcoreMesh(core_axis_name="c", subcore_axis_name="s")

    @functools.partial(
        pl.kernel,
        mesh=mesh,
        out_type=jax.ShapeDtypeStruct((n, _EMBED), jnp.float32),
        scratch_types=[
            pltpu.VMEM((npw,), jnp.int32),
            pltpu.VMEM((2, _RPC, _EMBED), jnp.float32),
            pltpu.SemaphoreType.DMA,
            pltpu.SemaphoreType.DMA,
        ],
        compiler_params=pltpu.CompilerParams(use_tc_tiling_on_sc=False),
    )
    def k(idx_hbm, table_hbm, out_hbm, idx_v, rows_v, gsem, wsem):
        wid = lax.axis_index("s") * info.num_cores + lax.axis_index("c")
        rbase = wid * npw
        pltpu.sync_copy(idx_hbm.at[pl.ds(rbase, npw)], idx_v)

        def gather(c, b):
            return pltpu.async_copy(
                table_hbm.at[idx_v.at[pl.ds(c * _RPC, _RPC)]],
                rows_v.at[b],
                gsem,
            )

        def writeback(c, b):
            return pltpu.async_copy(
                rows_v.at[b],
                out_hbm.at[pl.ds(rbase + c * _RPC, _RPC)],
                wsem,
            )

        gcp = [None] * nch
        wcp = [None] * nch
        gcp[0] = gather(0, 0)
        gcp[1] = gather(1, 1)
        for c in range(nch):
            gcp[c].wait()
            wcp[c] = writeback(c, c % 2)
            if c + 2 < nch:
                wcp[c].wait()
                gcp[c + 2] = gather(c + 2, c % 2)
        wcp[nch - 2].wait()
        wcp[nch - 1].wait()

    return k(idx, table)


def kernel(tensor, table):
    b, f = tensor.shape
    idx = tensor.reshape(-1).astype(jnp.int32)
    out = _flat_gather(idx, table)
    return out.reshape(b, f, _EMBED)
